# baseline (speedup 1.0000x reference)
import jax
import jax.numpy as jnp
from jax.experimental import pallas as pl
from jax.experimental.pallas import tpu as pltpu

TAU = 2.0
V_TH = 1.0

S_TILE = 256


def _ssm_kernel(x_ref, At_ref, Bt_ref, Ct_ref, D_ref, out_ref):
    T = x_ref.shape[1]
    s_tile = x_ref.shape[2]
    d_state = At_ref.shape[0]
    At = At_ref[...]
    Bt = Bt_ref[...]
    Ct = Ct_ref[...]
    D = D_ref[...]

    d_model = x_ref.shape[3]
    h = jnp.zeros((s_tile, d_state), dtype=jnp.float32)
    vs = jnp.zeros((s_tile, d_state), dtype=jnp.float32)
    vo = jnp.zeros((s_tile, d_model), dtype=jnp.float32)

    xu = jnp.dot(x_ref[0].reshape(T * s_tile, d_model), Bt,
                 preferred_element_type=jnp.float32)

    for t in range(T):
        xt = x_ref[0, t]
        su = (jnp.dot(h, At, preferred_element_type=jnp.float32)
              + xu[t * s_tile:(t + 1) * s_tile])
        vs = vs + (su - vs) / TAU
        s = (vs >= V_TH).astype(jnp.float32)
        vs = vs * (1.0 - s)
        ou = jnp.dot(s, Ct, preferred_element_type=jnp.float32) + xt + D
        vo = vo + (ou - vo) / TAU
        so = (vo >= V_TH).astype(jnp.float32)
        vo = vo * (1.0 - so)
        out_ref[0, t] = so
        h = s


def kernel(x, A, B, C, D):
    Bsz, T, S, d_model = x.shape
    d_state = A.shape[0]
    At = A.T
    Bt = B.T
    Ct = C.T
    D2 = D.reshape(1, d_model)

    grid = (Bsz, S // S_TILE)
    return pl.pallas_call(
        _ssm_kernel,
        grid=grid,
        in_specs=[
            pl.BlockSpec((1, T, S_TILE, d_model), lambda b, s: (b, 0, s, 0)),
            pl.BlockSpec((d_state, d_state), lambda b, s: (0, 0)),
            pl.BlockSpec((d_model, d_state), lambda b, s: (0, 0)),
            pl.BlockSpec((d_state, d_model), lambda b, s: (0, 0)),
            pl.BlockSpec((1, d_model), lambda b, s: (0, 0)),
        ],
        out_specs=pl.BlockSpec((1, T, S_TILE, d_model), lambda b, s: (b, 0, s, 0)),
        out_shape=jax.ShapeDtypeStruct((Bsz, T, S, d_model), jnp.float32),
        compiler_params=pltpu.CompilerParams(
            dimension_semantics=("parallel", "parallel"),
            vmem_limit_bytes=56 * 1024 * 1024,
        ),
    )(x, At, Bt, Ct, D2)


# speedup vs baseline: 1.0534x; 1.0534x over previous
import jax
import jax.numpy as jnp
from jax.experimental import pallas as pl
from jax.experimental.pallas import tpu as pltpu

TAU = 2.0
V_TH = 1.0

S_TILE = 256


def _ssm_kernel(x_ref, At_ref, Bt_ref, Ct_ref, D_ref, out_ref):
    T = x_ref.shape[1]
    s_tile = x_ref.shape[2]
    d_state = At_ref.shape[0]
    At = At_ref[...]
    Bt = Bt_ref[...]
    Ct = Ct_ref[...]
    D = D_ref[...]

    d_model = x_ref.shape[3]
    h = jnp.zeros((s_tile, d_state), dtype=jnp.float32)
    vs = jnp.zeros((s_tile, d_state), dtype=jnp.float32)
    vo = jnp.zeros((s_tile, d_model), dtype=jnp.float32)

    for t in range(T):
        xt = x_ref[0, t]
        su = (jnp.dot(h, At, preferred_element_type=jnp.float32)
              + jnp.dot(xt, Bt, preferred_element_type=jnp.float32))
        vs = vs + (su - vs) / TAU
        s = (vs >= V_TH).astype(jnp.float32)
        vs = vs * (1.0 - s)
        ou = jnp.dot(s, Ct, preferred_element_type=jnp.float32) + xt + D
        vo = vo + (ou - vo) / TAU
        so = (vo >= V_TH).astype(jnp.float32)
        vo = vo * (1.0 - so)
        out_ref[0, t] = so
        h = s


def kernel(x, A, B, C, D):
    Bsz, T, S, d_model = x.shape
    d_state = A.shape[0]
    At = A.T
    Bt = B.T
    Ct = C.T
    D2 = D.reshape(1, d_model)

    grid = (Bsz, S // S_TILE)
    return pl.pallas_call(
        _ssm_kernel,
        grid=grid,
        in_specs=[
            pl.BlockSpec((1, T, S_TILE, d_model), lambda b, s: (b, 0, s, 0)),
            pl.BlockSpec((d_state, d_state), lambda b, s: (0, 0)),
            pl.BlockSpec((d_model, d_state), lambda b, s: (0, 0)),
            pl.BlockSpec((d_state, d_model), lambda b, s: (0, 0)),
            pl.BlockSpec((1, d_model), lambda b, s: (0, 0)),
        ],
        out_specs=pl.BlockSpec((1, T, S_TILE, d_model), lambda b, s: (b, 0, s, 0)),
        out_shape=jax.ShapeDtypeStruct((Bsz, T, S, d_model), jnp.float32),
        compiler_params=pltpu.CompilerParams(
            dimension_semantics=("parallel", "parallel"),
            vmem_limit_bytes=56 * 1024 * 1024,
        ),
    )(x, At, Bt, Ct, D2)


# speedup vs baseline: 1.0562x; 1.0027x over previous
import jax
import jax.numpy as jnp
from jax.experimental import pallas as pl
from jax.experimental.pallas import tpu as pltpu

TAU = 2.0
V_TH = 1.0

S_TILE = 256


def _ssm_kernel(x_ref, At_ref, Bt_ref, Ct_ref, D_ref, out_ref):
    T = x_ref.shape[1]
    s_tile = x_ref.shape[2]
    d_state = At_ref.shape[0]
    At = At_ref[...]
    Bt = Bt_ref[...]
    Ct = Ct_ref[...]
    D = D_ref[...]

    d_model = x_ref.shape[3]
    h = jnp.zeros((s_tile, d_state), dtype=jnp.float32)
    vs = jnp.zeros((s_tile, d_state), dtype=jnp.float32)
    vo = jnp.zeros((s_tile, d_model), dtype=jnp.float32)

    for t in range(T):
        xt = x_ref[0, t]
        su = (jnp.dot(h, At, preferred_element_type=jnp.float32)
              + jnp.dot(xt, Bt, preferred_element_type=jnp.float32))
        vs = vs + (su - vs) * 0.5
        ms = vs >= V_TH
        s = ms.astype(jnp.float32)
        vs = jnp.where(ms, 0.0, vs)
        ou = jnp.dot(s, Ct, preferred_element_type=jnp.float32) + xt + D
        vo = vo + (ou - vo) * 0.5
        mo = vo >= V_TH
        so = mo.astype(jnp.float32)
        vo = jnp.where(mo, 0.0, vo)
        out_ref[0, t] = so
        h = s


def kernel(x, A, B, C, D):
    Bsz, T, S, d_model = x.shape
    d_state = A.shape[0]
    At = A.T
    Bt = B.T
    Ct = C.T
    D2 = D.reshape(1, d_model)

    grid = (Bsz, S // S_TILE)
    return pl.pallas_call(
        _ssm_kernel,
        grid=grid,
        in_specs=[
            pl.BlockSpec((1, T, S_TILE, d_model), lambda b, s: (b, 0, s, 0)),
            pl.BlockSpec((d_state, d_state), lambda b, s: (0, 0)),
            pl.BlockSpec((d_model, d_state), lambda b, s: (0, 0)),
            pl.BlockSpec((d_state, d_model), lambda b, s: (0, 0)),
            pl.BlockSpec((1, d_model), lambda b, s: (0, 0)),
        ],
        out_specs=pl.BlockSpec((1, T, S_TILE, d_model), lambda b, s: (b, 0, s, 0)),
        out_shape=jax.ShapeDtypeStruct((Bsz, T, S, d_model), jnp.float32),
        compiler_params=pltpu.CompilerParams(
            dimension_semantics=("parallel", "parallel"),
            vmem_limit_bytes=56 * 1024 * 1024,
        ),
    )(x, At, Bt, Ct, D2)


# speedup vs baseline: 1.1086x; 1.0496x over previous
import jax
import jax.numpy as jnp
from jax.experimental import pallas as pl
from jax.experimental.pallas import tpu as pltpu

TAU = 2.0
V_TH = 1.0

S_TILE = 512
T_CHUNK = 8


def _ssm_kernel(x_ref, At_ref, Bt_ref, Ct_ref, D_ref, out_ref,
                h_ref, vs_ref, vo_ref):
    t2 = pl.program_id(2)

    @pl.when(t2 == 0)
    def _init():
        h_ref[...] = jnp.zeros_like(h_ref)
        vs_ref[...] = jnp.zeros_like(vs_ref)
        vo_ref[...] = jnp.zeros_like(vo_ref)

    At = At_ref[...]
    Bt = Bt_ref[...]
    Ct = Ct_ref[...]
    D = D_ref[...]

    h = h_ref[...]
    vs = vs_ref[...]
    vo = vo_ref[...]

    for tt in range(T_CHUNK):
        xt = x_ref[0, tt]
        su = (jnp.dot(h, At, preferred_element_type=jnp.float32)
              + jnp.dot(xt, Bt, preferred_element_type=jnp.float32))
        vs = vs + (su - vs) * 0.5
        ms = vs >= V_TH
        s = ms.astype(jnp.float32)
        vs = jnp.where(ms, 0.0, vs)
        ou = jnp.dot(s, Ct, preferred_element_type=jnp.float32) + xt + D
        vo = vo + (ou - vo) * 0.5
        mo = vo >= V_TH
        so = mo.astype(jnp.float32)
        vo = jnp.where(mo, 0.0, vo)
        out_ref[0, tt] = so
        h = s

    h_ref[...] = h
    vs_ref[...] = vs
    vo_ref[...] = vo


def kernel(x, A, B, C, D):
    Bsz, T, S, d_model = x.shape
    d_state = A.shape[0]
    At = A.T
    Bt = B.T
    Ct = C.T
    D2 = D.reshape(1, d_model)

    grid = (Bsz, S // S_TILE, T // T_CHUNK)
    return pl.pallas_call(
        _ssm_kernel,
        grid=grid,
        in_specs=[
            pl.BlockSpec((1, T_CHUNK, S_TILE, d_model),
                         lambda b, s, t: (b, t, s, 0)),
            pl.BlockSpec((d_state, d_state), lambda b, s, t: (0, 0)),
            pl.BlockSpec((d_model, d_state), lambda b, s, t: (0, 0)),
            pl.BlockSpec((d_state, d_model), lambda b, s, t: (0, 0)),
            pl.BlockSpec((1, d_model), lambda b, s, t: (0, 0)),
        ],
        out_specs=pl.BlockSpec((1, T_CHUNK, S_TILE, d_model),
                               lambda b, s, t: (b, t, s, 0)),
        out_shape=jax.ShapeDtypeStruct((Bsz, T, S, d_model), jnp.float32),
        scratch_shapes=[
            pltpu.VMEM((S_TILE, d_state), jnp.float32),
            pltpu.VMEM((S_TILE, d_state), jnp.float32),
            pltpu.VMEM((S_TILE, d_model), jnp.float32),
        ],
        compiler_params=pltpu.CompilerParams(
            dimension_semantics=("parallel", "parallel", "arbitrary"),
            vmem_limit_bytes=56 * 1024 * 1024,
        ),
    )(x, At, Bt, Ct, D2)


# speedup vs baseline: 1.1207x; 1.0109x over previous
import jax
import jax.numpy as jnp
from jax.experimental import pallas as pl
from jax.experimental.pallas import tpu as pltpu

TAU = 2.0
V_TH = 1.0

S_TILE = 1024
T_CHUNK = 4


def _ssm_kernel(x_ref, At_ref, Bt_ref, Ct_ref, D_ref, out_ref,
                h_ref, vs_ref, vo_ref):
    t2 = pl.program_id(2)

    @pl.when(t2 == 0)
    def _init():
        h_ref[...] = jnp.zeros_like(h_ref)
        vs_ref[...] = jnp.zeros_like(vs_ref)
        vo_ref[...] = jnp.zeros_like(vo_ref)

    At = At_ref[...]
    Bt = Bt_ref[...]
    Ct = Ct_ref[...]
    D = D_ref[...]

    h = h_ref[...]
    vs = vs_ref[...]
    vo = vo_ref[...]

    for tt in range(T_CHUNK):
        xt = x_ref[0, tt]
        su = (jnp.dot(h, At, preferred_element_type=jnp.float32)
              + jnp.dot(xt, Bt, preferred_element_type=jnp.float32))
        vs = vs + (su - vs) * 0.5
        ms = vs >= V_TH
        s = ms.astype(jnp.float32)
        vs = jnp.where(ms, 0.0, vs)
        ou = jnp.dot(s, Ct, preferred_element_type=jnp.float32) + xt + D
        vo = vo + (ou - vo) * 0.5
        mo = vo >= V_TH
        so = mo.astype(jnp.float32)
        vo = jnp.where(mo, 0.0, vo)
        out_ref[0, tt] = so
        h = s

    h_ref[...] = h
    vs_ref[...] = vs
    vo_ref[...] = vo


def kernel(x, A, B, C, D):
    Bsz, T, S, d_model = x.shape
    d_state = A.shape[0]
    At = A.T
    Bt = B.T
    Ct = C.T
    D2 = D.reshape(1, d_model)

    grid = (Bsz, S // S_TILE, T // T_CHUNK)
    return pl.pallas_call(
        _ssm_kernel,
        grid=grid,
        in_specs=[
            pl.BlockSpec((1, T_CHUNK, S_TILE, d_model),
                         lambda b, s, t: (b, t, s, 0)),
            pl.BlockSpec((d_state, d_state), lambda b, s, t: (0, 0)),
            pl.BlockSpec((d_model, d_state), lambda b, s, t: (0, 0)),
            pl.BlockSpec((d_state, d_model), lambda b, s, t: (0, 0)),
            pl.BlockSpec((1, d_model), lambda b, s, t: (0, 0)),
        ],
        out_specs=pl.BlockSpec((1, T_CHUNK, S_TILE, d_model),
                               lambda b, s, t: (b, t, s, 0)),
        out_shape=jax.ShapeDtypeStruct((Bsz, T, S, d_model), jnp.float32),
        scratch_shapes=[
            pltpu.VMEM((S_TILE, d_state), jnp.float32),
            pltpu.VMEM((S_TILE, d_state), jnp.float32),
            pltpu.VMEM((S_TILE, d_model), jnp.float32),
        ],
        compiler_params=pltpu.CompilerParams(
            dimension_semantics=("parallel", "parallel", "arbitrary"),
            vmem_limit_bytes=56 * 1024 * 1024,
        ),
    )(x, At, Bt, Ct, D2)
